# Initial kernel scaffold; baseline (speedup 1.0000x reference)
import jax
import jax.numpy as jnp
from jax.experimental import pallas as pl


def kernel(x, edge_index, W1, b1, W2, b2, W3, b3, attn_in_w, attn_in_b, attn_out_w, attn_out_b, lin_w, lin_b, ln_g, ln_b, hyp_w, hyp_b):
    raise NotImplementedError("write your pallas kernel here")


# speedup vs baseline: 7.5076x; 7.5076x over previous
import functools
import jax
import jax.numpy as jnp
from jax import lax
from jax.experimental import pallas as pl
from jax.experimental.pallas import tpu as pltpu
from jax.experimental.pallas import tpu_sc as plsc

_N = 10000
_NP = 10240
_H = 128
_HEADS = 4
_DH = 32
_NTILES = 32
_GROUP = 128
_BI = 400


def _sc_degree(half, dst_g, groups):
    rows_per_tile = _NP // 16
    mesh = plsc.VectorSubcoreMesh(core_axis_name="c", subcore_axis_name="s")

    @functools.partial(
        pl.kernel,
        out_type=jax.ShapeDtypeStruct((2, 1, _NP), jnp.float32),
        mesh=mesh,
        scratch_types=[
            pltpu.VMEM((groups, _GROUP), jnp.int32),
            pltpu.VMEM((_GROUP,), jnp.float32),
            pltpu.VMEM_SHARED((_NP,), jnp.float32),
        ],
    )
    def k(half_hbm, dst_hbm, out_hbm, dst_v, ones_v, acc):
        c = lax.axis_index("c")
        s = lax.axis_index("s")
        wid = s * 2 + c
        base = s * rows_per_tile
        pltpu.sync_copy(half_hbm.at[pl.ds(base, rows_per_tile)],
                        acc.at[pl.ds(base, rows_per_tile)])
        pltpu.sync_copy(dst_hbm.at[wid], dst_v)
        one = jnp.full((16,), 1.0, jnp.float32)
        for i in range(_GROUP // 16):
            ones_v[pl.ds(i * 16, 16)] = one
        plsc.subcore_barrier()

        def body(j, carry):
            pltpu.sync_copy(ones_v, acc.at[dst_v.at[j]], add=True)
            return carry

        lax.fori_loop(0, groups, body, 0)
        plsc.subcore_barrier()
        pltpu.sync_copy(acc.at[pl.ds(base, rows_per_tile)],
                        out_hbm.at[c, 0, pl.ds(base, rows_per_tile)])

    return k(half, dst_g)


def _sc_scatter(feat, src_g, dst_g, width, groups):
    rows_per_tile = _NP // 16
    mesh = plsc.VectorSubcoreMesh(core_axis_name="c", subcore_axis_name="s")

    @functools.partial(
        pl.kernel,
        out_type=jax.ShapeDtypeStruct((2, _NP, width), jnp.float32),
        mesh=mesh,
        scratch_types=[
            pltpu.VMEM((groups, _GROUP), jnp.int32),
            pltpu.VMEM((groups, _GROUP), jnp.int32),
            pltpu.VMEM((_GROUP, width), jnp.float32),
            pltpu.VMEM_SHARED((_NP, width), jnp.float32),
            pltpu.SemaphoreType.DMA,
        ],
    )
    def k(feat_hbm, src_hbm, dst_hbm, out_hbm, src_v, dst_v, rows_v, acc, sem):
        c = lax.axis_index("c")
        s = lax.axis_index("s")
        wid = s * 2 + c
        base = s * rows_per_tile
        pltpu.sync_copy(feat_hbm.at[pl.ds(base, rows_per_tile)],
                        acc.at[pl.ds(base, rows_per_tile)])
        pltpu.sync_copy(src_hbm.at[wid], src_v)
        pltpu.sync_copy(dst_hbm.at[wid], dst_v)
        plsc.subcore_barrier()

        def body(j, carry):
            pltpu.async_copy(feat_hbm.at[src_v.at[j]], rows_v, sem).wait()
            pltpu.sync_copy(rows_v, acc.at[dst_v.at[j]], add=True)
            return carry

        lax.fori_loop(0, groups, body, 0)
        plsc.subcore_barrier()
        pltpu.sync_copy(acc.at[pl.ds(base, rows_per_tile)],
                        out_hbm.at[c, pl.ds(base, rows_per_tile)])

    return k(feat, src_g, dst_g)


def _dis_of(degp_ref):
    deg = degp_ref[0] + degp_ref[1]
    return lax.rsqrt(jnp.maximum(deg, 1.0))


def _gelu_exact(x):
    return x * 0.5 * (1.0 + lax.erf(x * (2.0 ** -0.5)))


def _rowmm(a, w):
    return lax.dot_general(a, w, (((1,), (1,)), ((), ())),
                           preferred_element_type=jnp.float32)


def _t0_body(x_ref, w_ref, degp_ref, xs_ref):
    dis = _dis_of(degp_ref)
    xs_ref[...] = _rowmm(x_ref[...], w_ref[...]) * dis


def _combine_body(p_ref, xs_ref, degp_ref, b_ref, w_ref, out_ref, *, act):
    dis = _dis_of(degp_ref)
    pre = (p_ref[0] + p_ref[1] - xs_ref[...]) * dis + b_ref[...]
    h = _gelu_exact(pre) if act else pre
    out_ref[...] = _rowmm(h, w_ref[...]) * dis


def _qkv_body(p_ref, xs_ref, degp_ref, b_ref, w_ref, wb_ref, out_ref):
    dis = _dis_of(degp_ref)
    h3 = (p_ref[0] + p_ref[1] - xs_ref[...]) * dis + b_ref[...]
    out_ref[...] = _rowmm(h3, w_ref[...]) + wb_ref[...]


def _attn_body(q_ref, k_ref, cw_ref):
    q = q_ref[0] * (_DH ** -0.5)
    kk = k_ref[0]
    s = lax.dot_general(q, kk, (((1,), (1,)), ((), ())),
                        preferred_element_type=jnp.float32)
    m = jnp.max(s, axis=1, keepdims=True)
    p = jnp.exp(s - m)
    l = jnp.sum(p, axis=1, keepdims=True)
    cw = jnp.sum(p / l, axis=0, keepdims=True)

    @pl.when(pl.program_id(1) == 0)
    def _init():
        cw_ref[...] = jnp.zeros_like(cw_ref)

    cw_ref[0, 0:1, :] += cw


def _norm(x):
    return jnp.sqrt(jnp.sum(x * x, axis=-1, keepdims=True))


def _artanh(x):
    x = jnp.clip(x, -1.0 + 1e-7, 1.0 - 1e-7)
    return 0.5 * (jnp.log1p(x) - jnp.log1p(-x))


def _expmap0(u):
    n = jnp.maximum(_norm(u), 1e-15)
    return jnp.tanh(n) * u / n


def _proj(x):
    maxn = 1.0 - 1e-5
    n = jnp.maximum(_norm(x), 1e-15)
    return jnp.where(n > maxn, x / n * maxn, x)


def _tail_body(cw_ref, v_ref, ow_ref, ob_ref, lw_ref, lb_ref, g_ref, be_ref,
               hw_ref, hb_ref, def_ref, hg_ref):
    parts = []
    for h in range(_HEADS):
        cwh = cw_ref[h, 0:1, :]
        vh = v_ref[h]
        parts.append(lax.dot_general(
            cwh, vh, (((1,), (0,)), ((), ())),
            preferred_element_type=jnp.float32))
    omean = jnp.concatenate(parts, axis=1) * (1.0 / _N)
    hg = _rowmm(omean, ow_ref[...]) + ob_ref[...]
    hg_ref[...] = hg
    t = _rowmm(hg, lw_ref[...]) + lb_ref[...]
    mu = jnp.mean(t, axis=-1, keepdims=True)
    var = jnp.mean((t - mu) * (t - mu), axis=-1, keepdims=True)
    t = (t - mu) / jnp.sqrt(var + 1e-5) * g_ref[...] + be_ref[...]
    t = _gelu_exact(t)
    xh = _proj(_expmap0(t))
    xn = jnp.maximum(_norm(xh), 1e-15)
    mx = _rowmm(xh, hw_ref[...])
    mxn = jnp.maximum(_norm(mx), 1e-15)
    mv = jnp.tanh(mxn / xn * _artanh(xn)) * mx / mxn
    mv = _proj(mv)
    bh = _proj(_expmap0(hb_ref[...]))
    x2 = jnp.sum(mv * mv, axis=-1, keepdims=True)
    y2 = jnp.sum(bh * bh, axis=-1, keepdims=True)
    xy = jnp.sum(mv * bh, axis=-1, keepdims=True)
    num = (1.0 + 2.0 * xy + y2) * mv + (1.0 - x2) * bh
    den = 1.0 + 2.0 * xy + x2 * y2
    out = _proj(num / jnp.maximum(den, 1e-15))
    on = jnp.maximum(_norm(out), 1e-15)
    def_ref[...] = _artanh(on) * out / on


def kernel(x, edge_index, W1, b1, W2, b2, W3, b3, attn_in_w, attn_in_b,
           attn_out_w, attn_out_b, lin_w, lin_b, ln_g, ln_b, hyp_w, hyp_b):
    e = edge_index.shape[1]
    per = _NTILES * _GROUP
    groups = -(-e // per)
    pad = groups * per - e
    src_g = jnp.concatenate(
        [edge_index[0], jnp.full((pad,), _N, jnp.int32)]).reshape(
            _NTILES, groups, _GROUP)
    dst_g = jnp.concatenate(
        [edge_index[1], jnp.full((pad,), _N, jnp.int32)]).reshape(
            _NTILES, groups, _GROUP)

    x_pad = jnp.pad(x, ((0, _NP - _N), (0, 0)))
    half = jnp.full((_NP,), 0.5, jnp.float32)

    degp = _sc_degree(half, dst_g, groups).reshape(2, _NP, 1)

    xs1 = pl.pallas_call(
        _t0_body,
        out_shape=jax.ShapeDtypeStruct((_NP, _H), jnp.float32),
    )(x_pad, W1, degp)

    p1 = _sc_scatter(xs1, src_g, dst_g, _H, groups)
    xs2 = pl.pallas_call(
        functools.partial(_combine_body, act=True),
        out_shape=jax.ShapeDtypeStruct((_NP, _H), jnp.float32),
    )(p1, xs1, degp, b1.reshape(1, _H), W2)

    p2 = _sc_scatter(xs2, src_g, dst_g, _H, groups)
    xs3 = pl.pallas_call(
        functools.partial(_combine_body, act=True),
        out_shape=jax.ShapeDtypeStruct((_NP, _H), jnp.float32),
    )(p2, xs2, degp, b2.reshape(1, _H), W3)

    p3 = _sc_scatter(xs3, src_g, dst_g, _H, groups)
    qkv = pl.pallas_call(
        _qkv_body,
        out_shape=jax.ShapeDtypeStruct((_NP, 3 * _H), jnp.float32),
    )(p3, xs3, degp, b3.reshape(1, _H), attn_in_w, attn_in_b.reshape(1, 3 * _H))

    qkv = qkv[:_N]
    qh = qkv[:, 0:_H].reshape(_N, _HEADS, _DH).transpose(1, 0, 2)
    kh = qkv[:, _H:2 * _H].reshape(_N, _HEADS, _DH).transpose(1, 0, 2)
    vh = qkv[:, 2 * _H:3 * _H].reshape(_N, _HEADS, _DH).transpose(1, 0, 2)

    nblk = _N // _BI
    cw = pl.pallas_call(
        _attn_body,
        grid=(_HEADS, nblk),
        in_specs=[
            pl.BlockSpec((1, _BI, _DH), lambda h, i: (h, i, 0)),
            pl.BlockSpec((1, _N, _DH), lambda h, i: (h, 0, 0)),
        ],
        out_specs=pl.BlockSpec((1, 8, _N), lambda h, i: (h, 0, 0)),
        out_shape=jax.ShapeDtypeStruct((_HEADS, 8, _N), jnp.float32),
    )(qh, kh)

    deformation, h_global = pl.pallas_call(
        _tail_body,
        out_shape=(
            jax.ShapeDtypeStruct((1, hyp_w.shape[0]), jnp.float32),
            jax.ShapeDtypeStruct((1, _H), jnp.float32),
        ),
    )(cw, vh, attn_out_w, attn_out_b.reshape(1, _H),
      lin_w, lin_b.reshape(1, 2 * _H), ln_g.reshape(1, 2 * _H),
      ln_b.reshape(1, 2 * _H), hyp_w, hyp_b.reshape(1, hyp_w.shape[0]))

    return (deformation, h_global)
